# baseline (speedup 1.0000x reference)
import jax
import jax.numpy as jnp
from jax import lax
from jax.experimental import pallas as pl
from jax.experimental.pallas import tpu as pltpu, tpu_sc as plsc

EMBED_DIM = 32
NUM_EMB = 17
N_EDGES = 1600000
NUM_WORKERS = 32
CHUNK = 1280
N_CHUNKS = N_EDGES // CHUNK
BASE_CHUNKS = N_CHUNKS // NUM_WORKERS
EXTRA = N_CHUNKS - BASE_CHUNKS * NUM_WORKERS
MAX_PAIRS = (BASE_CHUNKS + 2) // 2
LANES = 16
GROUPS = CHUNK // LANES
SPLAT_OFF = NUM_EMB * EMBED_DIM
TABLE_WORDS = SPLAT_OFF + EMBED_DIM * LANES


def _sc_body(edge_hbm, table_hbm, out_hbm,
             table_v, src0, dst0, src1, dst1, rows0, rows1,
             sem_in0, sem_in1, sem_out0, sem_out1):
    wid = lax.axis_index("s") * 2 + lax.axis_index("c")
    n_w = jnp.where(wid < EXTRA, BASE_CHUNKS + 1, BASE_CHUNKS)
    pltpu.sync_copy(table_hbm, table_v)

    def fire_in(i, sv, dv, sem):
        base = (wid + i * NUM_WORKERS) * CHUNK
        pltpu.async_copy(edge_hbm.at[0, pl.ds(base, CHUNK)], sv, sem)
        pltpu.async_copy(edge_hbm.at[1, pl.ds(base, CHUNK)], dv, sem)

    def wait_in(i, sv, dv, sem):
        base = (wid + i * NUM_WORKERS) * CHUNK
        pltpu.make_async_copy(edge_hbm.at[0, pl.ds(base, CHUNK)], sv,
                              sem).wait()
        pltpu.make_async_copy(edge_hbm.at[1, pl.ds(base, CHUNK)], dv,
                              sem).wait()

    def fire_out(i, rows, sem):
        base = (wid + i * NUM_WORKERS) * CHUNK
        pltpu.async_copy(rows, out_hbm.at[:, pl.ds(base, CHUNK)], sem)

    def wait_out(i, rows, sem):
        base = (wid + i * NUM_WORKERS) * CHUNK
        pltpu.make_async_copy(rows, out_hbm.at[:, pl.ds(base, CHUNK)],
                              sem).wait()

    def take16(vec, idx):
        return lax.gather(
            vec, idx[:, None],
            lax.GatherDimensionNumbers(offset_dims=(),
                                       collapsed_slice_dims=(0,),
                                       start_index_map=(0,)),
            slice_sizes=(1,),
            mode=lax.GatherScatterMode.PROMISE_IN_BOUNDS)

    def compute(sv, dv, rows):
        tcol = [table_v[pl.ds(NUM_EMB * c, LANES)] for c in range(EMBED_DIM)]

        def group_body(g, c2):
            s = sv[pl.ds(g * LANES, LANES)]
            d = dv[pl.ds(g * LANES, LANES)]
            x = jnp.abs(s - d) - 1
            bits = plsc.bitcast(x.astype(jnp.float32), jnp.int32)
            code = jnp.where(x >= 1, (bits >> 23) - 126, 0)
            code15 = jnp.minimum(code, 15)
            m16 = code >= 16
            for c in range(EMBED_DIM):
                vmain = take16(tcol[c], code15)
                v16 = table_v[pl.ds(SPLAT_OFF + c * LANES, LANES)]
                rows[c, pl.ds(g * LANES, LANES)] = jnp.where(m16, v16, vmain)
            return c2
        lax.fori_loop(0, GROUPS, group_body, 0, unroll=2)

    fire_in(0, src0, dst0, sem_in0)
    fire_in(1, src1, dst1, sem_in1)

    def pair_body(k, carry):
        i0 = 2 * k
        wait_in(i0, src0, dst0, sem_in0)

        @pl.when(k > 0)
        def _():
            wait_out(i0 - 2, rows0, sem_out0)
        compute(src0, dst0, rows0)
        fire_out(i0, rows0, sem_out0)

        @pl.when(i0 + 2 < n_w)
        def _():
            fire_in(i0 + 2, src0, dst0, sem_in0)

        i1 = 2 * k + 1

        @pl.when(k > 0)
        def _():
            wait_out(i1 - 2, rows1, sem_out1)

        @pl.when(i1 < n_w)
        def _():
            wait_in(i1, src1, dst1, sem_in1)
            compute(src1, dst1, rows1)
            fire_out(i1, rows1, sem_out1)

            @pl.when(i1 + 2 < n_w)
            def _():
                fire_in(i1 + 2, src1, dst1, sem_in1)
        return carry

    lax.fori_loop(0, MAX_PAIRS, pair_body, 0)

    wait_out(n_w - 1, rows0, sem_out0)

    @pl.when(n_w == BASE_CHUNKS + 1)
    def _():
        wait_out(n_w - 1, rows1, sem_out1)


def kernel(edge_index, emb_weight):
    mesh = plsc.VectorSubcoreMesh(core_axis_name="c", subcore_axis_name="s")
    run = pl.kernel(
        _sc_body,
        out_type=jax.ShapeDtypeStruct((EMBED_DIM, N_EDGES), jnp.float32),
        mesh=mesh,
        scratch_types=[
            pltpu.VMEM((TABLE_WORDS,), jnp.float32),
            pltpu.VMEM((CHUNK,), jnp.int32),
            pltpu.VMEM((CHUNK,), jnp.int32),
            pltpu.VMEM((CHUNK,), jnp.int32),
            pltpu.VMEM((CHUNK,), jnp.int32),
            pltpu.VMEM((EMBED_DIM, CHUNK), jnp.float32),
            pltpu.VMEM((EMBED_DIM, CHUNK), jnp.float32),
            pltpu.SemaphoreType.DMA,
            pltpu.SemaphoreType.DMA,
            pltpu.SemaphoreType.DMA,
            pltpu.SemaphoreType.DMA,
        ],
        compiler_params=pltpu.CompilerParams(needs_layout_passes=False),
    )
    table = jnp.concatenate(
        [emb_weight.T.reshape(-1), jnp.repeat(emb_weight[NUM_EMB - 1], LANES)])
    out_t = run(edge_index, table)
    return out_t.T


# speedup vs baseline: 1.1254x; 1.1254x over previous
import jax
import jax.numpy as jnp
from jax import lax
from jax.experimental import pallas as pl
from jax.experimental.pallas import tpu as pltpu, tpu_sc as plsc

EMBED_DIM = 32
NUM_EMB = 17
N_EDGES = 1600000
NUM_WORKERS = 32
CHUNK = 1280
N_CHUNKS = N_EDGES // CHUNK
BASE_CHUNKS = N_CHUNKS // NUM_WORKERS
EXTRA = N_CHUNKS - BASE_CHUNKS * NUM_WORKERS
MAX_PAIRS = (BASE_CHUNKS + 2) // 2
LANES = 16
GROUPS = CHUNK // LANES
COL_STRIDE = 24
TABLE_WORDS = COL_STRIDE * EMBED_DIM


def _sc_body(edge_hbm, table_hbm, out_hbm,
             table_v, src0, dst0, src1, dst1, rows0, rows1,
             sem_in0, sem_in1, sem_out0, sem_out1):
    wid = lax.axis_index("s") * 2 + lax.axis_index("c")
    n_w = jnp.where(wid < EXTRA, BASE_CHUNKS + 1, BASE_CHUNKS)
    pltpu.sync_copy(table_hbm, table_v)

    def fire_in(i, sv, dv, sem):
        base = (wid + i * NUM_WORKERS) * CHUNK
        pltpu.async_copy(edge_hbm.at[0, pl.ds(base, CHUNK)], sv, sem)
        pltpu.async_copy(edge_hbm.at[1, pl.ds(base, CHUNK)], dv, sem)

    def wait_in(i, sv, dv, sem):
        base = (wid + i * NUM_WORKERS) * CHUNK
        pltpu.make_async_copy(edge_hbm.at[0, pl.ds(base, CHUNK)], sv,
                              sem).wait()
        pltpu.make_async_copy(edge_hbm.at[1, pl.ds(base, CHUNK)], dv,
                              sem).wait()

    def fire_out(i, rows, sem):
        base = (wid + i * NUM_WORKERS) * CHUNK
        pltpu.async_copy(rows, out_hbm.at[:, pl.ds(base, CHUNK)], sem)

    def wait_out(i, rows, sem):
        base = (wid + i * NUM_WORKERS) * CHUNK
        pltpu.make_async_copy(rows, out_hbm.at[:, pl.ds(base, CHUNK)],
                              sem).wait()

    def compute(sv, dv, rows):
        def group_body(g, c2):
            s = sv[pl.ds(g * LANES, LANES)]
            d = dv[pl.ds(g * LANES, LANES)]
            x = jnp.abs(s - d) - 1
            bits = plsc.bitcast(x.astype(jnp.float32), jnp.int32)
            code = jnp.where(x >= 1, (bits >> 23) - 126, 0)
            for c in range(EMBED_DIM):
                v = plsc.load_gather(
                    table_v.at[pl.ds(c * COL_STRIDE, NUM_EMB)], [code])
                rows[c, pl.ds(g * LANES, LANES)] = v
            return c2
        lax.fori_loop(0, GROUPS, group_body, 0, unroll=2)

    fire_in(0, src0, dst0, sem_in0)
    fire_in(1, src1, dst1, sem_in1)

    def pair_body(k, carry):
        i0 = 2 * k
        wait_in(i0, src0, dst0, sem_in0)

        @pl.when(k > 0)
        def _():
            wait_out(i0 - 2, rows0, sem_out0)
        compute(src0, dst0, rows0)
        fire_out(i0, rows0, sem_out0)

        @pl.when(i0 + 2 < n_w)
        def _():
            fire_in(i0 + 2, src0, dst0, sem_in0)

        i1 = 2 * k + 1

        @pl.when(k > 0)
        def _():
            wait_out(i1 - 2, rows1, sem_out1)

        @pl.when(i1 < n_w)
        def _():
            wait_in(i1, src1, dst1, sem_in1)
            compute(src1, dst1, rows1)
            fire_out(i1, rows1, sem_out1)

            @pl.when(i1 + 2 < n_w)
            def _():
                fire_in(i1 + 2, src1, dst1, sem_in1)
        return carry

    lax.fori_loop(0, MAX_PAIRS, pair_body, 0)

    wait_out(n_w - 1, rows0, sem_out0)

    @pl.when(n_w == BASE_CHUNKS + 1)
    def _():
        wait_out(n_w - 1, rows1, sem_out1)


def kernel(edge_index, emb_weight):
    mesh = plsc.VectorSubcoreMesh(core_axis_name="c", subcore_axis_name="s")
    run = pl.kernel(
        _sc_body,
        out_type=jax.ShapeDtypeStruct((EMBED_DIM, N_EDGES), jnp.float32),
        mesh=mesh,
        scratch_types=[
            pltpu.VMEM((TABLE_WORDS,), jnp.float32),
            pltpu.VMEM((CHUNK,), jnp.int32),
            pltpu.VMEM((CHUNK,), jnp.int32),
            pltpu.VMEM((CHUNK,), jnp.int32),
            pltpu.VMEM((CHUNK,), jnp.int32),
            pltpu.VMEM((EMBED_DIM, CHUNK), jnp.float32),
            pltpu.VMEM((EMBED_DIM, CHUNK), jnp.float32),
            pltpu.SemaphoreType.DMA,
            pltpu.SemaphoreType.DMA,
            pltpu.SemaphoreType.DMA,
            pltpu.SemaphoreType.DMA,
        ],
        compiler_params=pltpu.CompilerParams(needs_layout_passes=False),
    )
    table = jnp.pad(emb_weight.T,
                    ((0, 0), (0, COL_STRIDE - NUM_EMB))).reshape(-1)
    out_t = run(edge_index, table)
    return out_t.T


# speedup vs baseline: 2.6780x; 2.3796x over previous
import jax
import jax.numpy as jnp
from jax import lax
from jax.experimental import pallas as pl
from jax.experimental.pallas import tpu as pltpu, tpu_sc as plsc

EMBED_DIM = 32
NUM_EMB = 17
N_EDGES = 1600000
NUM_WORKERS = 32
CHUNK = 1280
N_CHUNKS = N_EDGES // CHUNK
BASE_CHUNKS = N_CHUNKS // NUM_WORKERS
EXTRA = N_CHUNKS - BASE_CHUNKS * NUM_WORKERS
MAX_PAIRS = (BASE_CHUNKS + 2) // 2
LANES = 16
GROUPS = CHUNK // LANES
COL_STRIDE = 24
TABLE_WORDS = COL_STRIDE * EMBED_DIM


def _sc_body(edge_hbm, table_hbm, out_hbm,
             table_v, src0, dst0, src1, dst1, rows0, rows1,
             sem_in0, sem_in1, sem_out0, sem_out1):
    wid = lax.axis_index("s") * 2 + lax.axis_index("c")
    n_w = jnp.where(wid < EXTRA, BASE_CHUNKS + 1, BASE_CHUNKS)
    pltpu.sync_copy(table_hbm, table_v)

    def fire_in(i, sv, dv, sem):
        base = (wid + i * NUM_WORKERS) * CHUNK
        pltpu.async_copy(edge_hbm.at[0, pl.ds(base, CHUNK)], sv, sem)
        pltpu.async_copy(edge_hbm.at[1, pl.ds(base, CHUNK)], dv, sem)

    def wait_in(i, sv, dv, sem):
        base = (wid + i * NUM_WORKERS) * CHUNK
        pltpu.make_async_copy(edge_hbm.at[0, pl.ds(base, CHUNK)], sv,
                              sem).wait()
        pltpu.make_async_copy(edge_hbm.at[1, pl.ds(base, CHUNK)], dv,
                              sem).wait()

    def fire_out(i, rows, sem):
        base = (wid + i * NUM_WORKERS) * CHUNK
        pltpu.async_copy(rows, out_hbm.at[:, pl.ds(base, CHUNK)], sem)

    def wait_out(i, rows, sem):
        base = (wid + i * NUM_WORKERS) * CHUNK
        pltpu.make_async_copy(rows, out_hbm.at[:, pl.ds(base, CHUNK)],
                              sem).wait()

    def compute(sv, dv, rows):
        def group_body(g, c2):
            s = sv[pl.ds(g * LANES, LANES)]
            d = dv[pl.ds(g * LANES, LANES)]
            x = jnp.abs(s - d) - 1
            bits = plsc.bitcast(x.astype(jnp.float32), jnp.int32)
            code = jnp.where(x >= 1, (bits >> 23) - 126, 0)
            for c0 in range(0, EMBED_DIM, 8):
                vs = [plsc.load_gather(
                          table_v.at[pl.ds(c * COL_STRIDE, NUM_EMB)], [code])
                      for c in range(c0, c0 + 8)]
                for j, c in enumerate(range(c0, c0 + 8)):
                    rows[c, pl.ds(g * LANES, LANES)] = vs[j]
            return c2
        lax.fori_loop(0, GROUPS, group_body, 0, unroll=2)

    fire_in(0, src0, dst0, sem_in0)
    fire_in(1, src1, dst1, sem_in1)

    def pair_body(k, carry):
        i0 = 2 * k
        wait_in(i0, src0, dst0, sem_in0)

        @pl.when(k > 0)
        def _():
            wait_out(i0 - 2, rows0, sem_out0)
        compute(src0, dst0, rows0)
        fire_out(i0, rows0, sem_out0)

        @pl.when(i0 + 2 < n_w)
        def _():
            fire_in(i0 + 2, src0, dst0, sem_in0)

        i1 = 2 * k + 1

        @pl.when(k > 0)
        def _():
            wait_out(i1 - 2, rows1, sem_out1)

        @pl.when(i1 < n_w)
        def _():
            wait_in(i1, src1, dst1, sem_in1)
            compute(src1, dst1, rows1)
            fire_out(i1, rows1, sem_out1)

            @pl.when(i1 + 2 < n_w)
            def _():
                fire_in(i1 + 2, src1, dst1, sem_in1)
        return carry

    lax.fori_loop(0, MAX_PAIRS, pair_body, 0)

    wait_out(n_w - 1, rows0, sem_out0)

    @pl.when(n_w == BASE_CHUNKS + 1)
    def _():
        wait_out(n_w - 1, rows1, sem_out1)


def kernel(edge_index, emb_weight):
    mesh = plsc.VectorSubcoreMesh(core_axis_name="c", subcore_axis_name="s")
    run = pl.kernel(
        _sc_body,
        out_type=jax.ShapeDtypeStruct((EMBED_DIM, N_EDGES), jnp.float32),
        mesh=mesh,
        scratch_types=[
            pltpu.VMEM((TABLE_WORDS,), jnp.float32),
            pltpu.VMEM((CHUNK,), jnp.int32),
            pltpu.VMEM((CHUNK,), jnp.int32),
            pltpu.VMEM((CHUNK,), jnp.int32),
            pltpu.VMEM((CHUNK,), jnp.int32),
            pltpu.VMEM((EMBED_DIM, CHUNK), jnp.float32),
            pltpu.VMEM((EMBED_DIM, CHUNK), jnp.float32),
            pltpu.SemaphoreType.DMA,
            pltpu.SemaphoreType.DMA,
            pltpu.SemaphoreType.DMA,
            pltpu.SemaphoreType.DMA,
        ],
        compiler_params=pltpu.CompilerParams(needs_layout_passes=False),
    )
    table = jnp.pad(emb_weight.T,
                    ((0, 0), (0, COL_STRIDE - NUM_EMB))).reshape(-1)
    out_t = run(edge_index, table)
    return out_t.T


# speedup vs baseline: 3.1653x; 1.1819x over previous
import jax
import jax.numpy as jnp
from jax import lax
from jax.experimental import pallas as pl
from jax.experimental.pallas import tpu as pltpu, tpu_sc as plsc

EMBED_DIM = 32
NUM_EMB = 17
N_EDGES = 1600000
NUM_WORKERS = 32
CHUNK = 1280
N_CHUNKS = N_EDGES // CHUNK
BASE_CHUNKS = N_CHUNKS // NUM_WORKERS
EXTRA = N_CHUNKS - BASE_CHUNKS * NUM_WORKERS
MAX_PAIRS = (BASE_CHUNKS + 2) // 2
LANES = 16
GROUPS = CHUNK // LANES
COL_STRIDE = 24
TABLE_WORDS = COL_STRIDE * EMBED_DIM


def _sc_body(edge_hbm, table_hbm, out_hbm,
             table_v, src0, dst0, src1, dst1, rows0, rows1,
             sem_in0, sem_in1, sem_out0, sem_out1):
    wid = lax.axis_index("s") * 2 + lax.axis_index("c")
    n_w = jnp.where(wid < EXTRA, BASE_CHUNKS + 1, BASE_CHUNKS)
    pltpu.sync_copy(table_hbm, table_v)

    def fire_in(i, sv, dv, sem):
        base = (wid + i * NUM_WORKERS) * CHUNK
        pltpu.async_copy(edge_hbm.at[0, pl.ds(base, CHUNK)], sv, sem)
        pltpu.async_copy(edge_hbm.at[1, pl.ds(base, CHUNK)], dv, sem)

    def wait_in(i, sv, dv, sem):
        base = (wid + i * NUM_WORKERS) * CHUNK
        pltpu.make_async_copy(edge_hbm.at[0, pl.ds(base, CHUNK)], sv,
                              sem).wait()
        pltpu.make_async_copy(edge_hbm.at[1, pl.ds(base, CHUNK)], dv,
                              sem).wait()

    def fire_out(i, rows, sem):
        base = (wid + i * NUM_WORKERS) * CHUNK
        pltpu.async_copy(rows, out_hbm.at[:, pl.ds(base, CHUNK)], sem)

    def wait_out(i, rows, sem):
        base = (wid + i * NUM_WORKERS) * CHUNK
        pltpu.make_async_copy(rows, out_hbm.at[:, pl.ds(base, CHUNK)],
                              sem).wait()

    def compute(sv, dv, rows):
        def group_body(g, c2):
            s = sv[pl.ds(g * LANES, LANES)]
            d = dv[pl.ds(g * LANES, LANES)]
            x = jnp.abs(s - d) - 1
            bits = plsc.bitcast(x.astype(jnp.float32), jnp.int32)
            code = jnp.where(x >= 1, (bits >> 23) - 126, 0)
            vs = [None] * EMBED_DIM
            for c in range(EMBED_DIM + 8):
                if c < EMBED_DIM:
                    vs[c] = plsc.load_gather(
                        table_v.at[pl.ds(c * COL_STRIDE, NUM_EMB)], [code])
                if c >= 8:
                    rows[c - 8, pl.ds(g * LANES, LANES)] = vs[c - 8]
            return c2
        lax.fori_loop(0, GROUPS, group_body, 0, unroll=2)

    fire_in(0, src0, dst0, sem_in0)
    fire_in(1, src1, dst1, sem_in1)

    def pair_body(k, carry):
        i0 = 2 * k
        wait_in(i0, src0, dst0, sem_in0)

        @pl.when(k > 0)
        def _():
            wait_out(i0 - 2, rows0, sem_out0)
        compute(src0, dst0, rows0)
        fire_out(i0, rows0, sem_out0)

        @pl.when(i0 + 2 < n_w)
        def _():
            fire_in(i0 + 2, src0, dst0, sem_in0)

        i1 = 2 * k + 1

        @pl.when(k > 0)
        def _():
            wait_out(i1 - 2, rows1, sem_out1)

        @pl.when(i1 < n_w)
        def _():
            wait_in(i1, src1, dst1, sem_in1)
            compute(src1, dst1, rows1)
            fire_out(i1, rows1, sem_out1)

            @pl.when(i1 + 2 < n_w)
            def _():
                fire_in(i1 + 2, src1, dst1, sem_in1)
        return carry

    lax.fori_loop(0, MAX_PAIRS, pair_body, 0)

    wait_out(n_w - 1, rows0, sem_out0)

    @pl.when(n_w == BASE_CHUNKS + 1)
    def _():
        wait_out(n_w - 1, rows1, sem_out1)


def kernel(edge_index, emb_weight):
    mesh = plsc.VectorSubcoreMesh(core_axis_name="c", subcore_axis_name="s")
    run = pl.kernel(
        _sc_body,
        out_type=jax.ShapeDtypeStruct((EMBED_DIM, N_EDGES), jnp.float32),
        mesh=mesh,
        scratch_types=[
            pltpu.VMEM((TABLE_WORDS,), jnp.float32),
            pltpu.VMEM((CHUNK,), jnp.int32),
            pltpu.VMEM((CHUNK,), jnp.int32),
            pltpu.VMEM((CHUNK,), jnp.int32),
            pltpu.VMEM((CHUNK,), jnp.int32),
            pltpu.VMEM((EMBED_DIM, CHUNK), jnp.float32),
            pltpu.VMEM((EMBED_DIM, CHUNK), jnp.float32),
            pltpu.SemaphoreType.DMA,
            pltpu.SemaphoreType.DMA,
            pltpu.SemaphoreType.DMA,
            pltpu.SemaphoreType.DMA,
        ],
        compiler_params=pltpu.CompilerParams(needs_layout_passes=False),
    )
    table = jnp.pad(emb_weight.T,
                    ((0, 0), (0, COL_STRIDE - NUM_EMB))).reshape(-1)
    out_t = run(edge_index, table)
    return out_t.T


# speedup vs baseline: 3.2951x; 1.0410x over previous
import jax
import jax.numpy as jnp
from jax import lax
from jax.experimental import pallas as pl
from jax.experimental.pallas import tpu as pltpu, tpu_sc as plsc

EMBED_DIM = 32
NUM_EMB = 17
N_EDGES = 1600000
NUM_WORKERS = 32
CHUNK = 1280
N_CHUNKS = N_EDGES // CHUNK
BASE_CHUNKS = N_CHUNKS // NUM_WORKERS
EXTRA = N_CHUNKS - BASE_CHUNKS * NUM_WORKERS
MAX_PAIRS = (BASE_CHUNKS + 2) // 2
LANES = 16
GROUPS = CHUNK // LANES
COL_STRIDE = 24
TABLE_WORDS = COL_STRIDE * EMBED_DIM


def _sc_body(edge_hbm, table_hbm, out_hbm,
             table_v, src0, dst0, src1, dst1, rows0, rows1,
             sem_in0, sem_in1, sem_out0, sem_out1):
    wid = lax.axis_index("s") * 2 + lax.axis_index("c")
    n_w = jnp.where(wid < EXTRA, BASE_CHUNKS + 1, BASE_CHUNKS)
    pltpu.sync_copy(table_hbm, table_v)

    def fire_in(i, sv, dv, sem):
        base = (wid + i * NUM_WORKERS) * CHUNK
        pltpu.async_copy(edge_hbm.at[0, pl.ds(base, CHUNK)], sv, sem)
        pltpu.async_copy(edge_hbm.at[1, pl.ds(base, CHUNK)], dv, sem)

    def wait_in(i, sv, dv, sem):
        base = (wid + i * NUM_WORKERS) * CHUNK
        pltpu.make_async_copy(edge_hbm.at[0, pl.ds(base, CHUNK)], sv,
                              sem).wait()
        pltpu.make_async_copy(edge_hbm.at[1, pl.ds(base, CHUNK)], dv,
                              sem).wait()

    def fire_out(i, rows, sem):
        base = (wid + i * NUM_WORKERS) * CHUNK
        pltpu.async_copy(rows, out_hbm.at[:, pl.ds(base, CHUNK)], sem)

    def wait_out(i, rows, sem):
        base = (wid + i * NUM_WORKERS) * CHUNK
        pltpu.make_async_copy(rows, out_hbm.at[:, pl.ds(base, CHUNK)],
                              sem).wait()

    def compute(sv, dv, rows):
        def group_body(g, c2):
            s = sv[pl.ds(g * LANES, LANES)]
            d = dv[pl.ds(g * LANES, LANES)]
            x = jnp.abs(s - d) - 1
            bits = plsc.bitcast(x.astype(jnp.float32), jnp.int32)
            code = jnp.where(x >= 1, (bits >> 23) - 126, 0)
            vs = [None] * EMBED_DIM
            for c in range(EMBED_DIM + 8):
                if c < EMBED_DIM:
                    vs[c] = plsc.load_gather(
                        table_v.at[pl.ds(c * COL_STRIDE, NUM_EMB)], [code])
                if c >= 8:
                    rows[c - 8, pl.ds(g * LANES, LANES)] = vs[c - 8]
            return c2
        lax.fori_loop(0, GROUPS, group_body, 0, unroll=4)

    fire_in(0, src0, dst0, sem_in0)
    fire_in(1, src1, dst1, sem_in1)

    def pair_body(k, carry):
        i0 = 2 * k
        wait_in(i0, src0, dst0, sem_in0)

        @pl.when(k > 0)
        def _():
            wait_out(i0 - 2, rows0, sem_out0)
        compute(src0, dst0, rows0)
        fire_out(i0, rows0, sem_out0)

        @pl.when(i0 + 2 < n_w)
        def _():
            fire_in(i0 + 2, src0, dst0, sem_in0)

        i1 = 2 * k + 1

        @pl.when(k > 0)
        def _():
            wait_out(i1 - 2, rows1, sem_out1)

        @pl.when(i1 < n_w)
        def _():
            wait_in(i1, src1, dst1, sem_in1)
            compute(src1, dst1, rows1)
            fire_out(i1, rows1, sem_out1)

            @pl.when(i1 + 2 < n_w)
            def _():
                fire_in(i1 + 2, src1, dst1, sem_in1)
        return carry

    lax.fori_loop(0, MAX_PAIRS, pair_body, 0)

    wait_out(n_w - 1, rows0, sem_out0)

    @pl.when(n_w == BASE_CHUNKS + 1)
    def _():
        wait_out(n_w - 1, rows1, sem_out1)


def kernel(edge_index, emb_weight):
    mesh = plsc.VectorSubcoreMesh(core_axis_name="c", subcore_axis_name="s")
    run = pl.kernel(
        _sc_body,
        out_type=jax.ShapeDtypeStruct((EMBED_DIM, N_EDGES), jnp.float32),
        mesh=mesh,
        scratch_types=[
            pltpu.VMEM((TABLE_WORDS,), jnp.float32),
            pltpu.VMEM((CHUNK,), jnp.int32),
            pltpu.VMEM((CHUNK,), jnp.int32),
            pltpu.VMEM((CHUNK,), jnp.int32),
            pltpu.VMEM((CHUNK,), jnp.int32),
            pltpu.VMEM((EMBED_DIM, CHUNK), jnp.float32),
            pltpu.VMEM((EMBED_DIM, CHUNK), jnp.float32),
            pltpu.SemaphoreType.DMA,
            pltpu.SemaphoreType.DMA,
            pltpu.SemaphoreType.DMA,
            pltpu.SemaphoreType.DMA,
        ],
        compiler_params=pltpu.CompilerParams(needs_layout_passes=False),
    )
    table = jnp.pad(emb_weight.T,
                    ((0, 0), (0, COL_STRIDE - NUM_EMB))).reshape(-1)
    out_t = run(edge_index, table)
    return out_t.T
